# Initial kernel scaffold; baseline (speedup 1.0000x reference)
import functools

import jax
import jax.numpy as jnp
from jax import lax
from jax.experimental import pallas as pl
from jax.experimental.pallas import tpu as pltpu
from jax.experimental.pallas import tpu_sc as plsc

NC = 2
NS = 16
NW = NC * NS
LANES = 16


def _make_kernel(B, K, D, BC):
    BPW = B // NW
    NCHUNK = BPW // BC
    NIDX = BC * K

    win_starts = list(range(0, D - LANES + 1, LANES))
    tail_keep = D - (win_starts[-1] + LANES)
    if tail_keep > 0:
        win_starts.append(D - LANES)
    NWIN = len(win_starts)

    GPIECE = 128
    n_pieces = max(1, (NIDX + GPIECE - 1) // GPIECE)

    mesh = plsc.VectorSubcoreMesh(core_axis_name="c", subcore_axis_name="s")

    @functools.partial(
        pl.kernel,
        out_type=[
            jax.ShapeDtypeStruct((B,), jnp.float32),
            jax.ShapeDtypeStruct((B * K,), jnp.float32),
        ],
        mesh=mesh,
        scratch_types=[
            pltpu.VMEM((BC,), jnp.int32),
            pltpu.VMEM((BC,), jnp.int32),
            pltpu.VMEM((NIDX,), jnp.int32),
            pltpu.VMEM((BC, D), jnp.float32),
            pltpu.VMEM((BC, D), jnp.float32),
            pltpu.VMEM((NIDX, D), jnp.float32),
            pltpu.VMEM((BC,), jnp.float32),
            pltpu.VMEM((NIDX,), jnp.float32),
            pltpu.SemaphoreType.DMA,
        ],
    )
    def k(c_hbm, p_hbm, n_hbm, wc_hbm, wx_hbm, pos_out, neg_out,
          cidx, pidx, nidx, cbuf, pbuf, nbuf, pos_stage, neg_stage, sem):
        wid = lax.axis_index("s") * NC + lax.axis_index("c")
        base_b = wid * BPW
        lane = lax.iota(jnp.int32, LANES)
        tail_mask = lane >= (LANES - tail_keep)

        def chunk_body(ci, carry):
            b0 = base_b + ci * BC
            pltpu.sync_copy(c_hbm.at[pl.ds(b0, BC)], cidx)
            pltpu.sync_copy(p_hbm.at[pl.ds(b0, BC)], pidx)
            pltpu.sync_copy(n_hbm.at[pl.ds(b0 * K, NIDX)], nidx)

            copies = [
                pltpu.async_copy(wc_hbm.at[cidx], cbuf, sem),
                pltpu.async_copy(wx_hbm.at[pidx], pbuf, sem),
            ]
            for g in range(n_pieces):
                lo = g * GPIECE
                sz = min(GPIECE, NIDX - lo)
                copies.append(pltpu.async_copy(
                    wx_hbm.at[nidx.at[pl.ds(lo, sz)]],
                    nbuf.at[pl.ds(lo, sz), :], sem))
            for cpy in copies:
                cpy.wait()

            def b_body(b, carry2):
                cw = []
                for wi, ws in enumerate(win_starts):
                    v = cbuf[b, pl.ds(ws, LANES)]
                    if tail_keep > 0 and wi == NWIN - 1:
                        v = jnp.where(tail_mask, v, 0.0)
                    cw.append(v)
                acc = cw[0] * pbuf[b, pl.ds(win_starts[0], LANES)]
                for wi in range(1, NWIN):
                    acc = acc + cw[wi] * pbuf[b, pl.ds(win_starts[wi], LANES)]
                pos_stage[b] = jnp.sum(acc)
                for kk in range(K):
                    r = b * K + kk
                    acc = cw[0] * nbuf[r, pl.ds(win_starts[0], LANES)]
                    for wi in range(1, NWIN):
                        acc = acc + cw[wi] * nbuf[r, pl.ds(win_starts[wi], LANES)]
                    neg_stage[r] = jnp.sum(acc)
                return carry2

            lax.fori_loop(0, BC, b_body, 0)
            pltpu.sync_copy(pos_stage, pos_out.at[pl.ds(b0, BC)])
            pltpu.sync_copy(neg_stage, neg_out.at[pl.ds(b0 * K, NIDX)])
            return carry

        lax.fori_loop(0, NCHUNK, chunk_body, 0)

    return k


def kernel(center, pos_context, neg_contexts, W_center, W_context):
    B = center.shape[0]
    K = neg_contexts.shape[1]
    D = W_center.shape[1]
    c = center.astype(jnp.int32)
    p = pos_context.astype(jnp.int32)
    n = neg_contexts.astype(jnp.int32).reshape(-1)
    wc = W_center.astype(jnp.float32)
    wx = W_context.astype(jnp.float32)

    k = _make_kernel(B, K, D, BC=32)
    pos_flat, neg_flat = k(c, p, n, wc, wx)
    return pos_flat.reshape(B, 1), neg_flat.reshape(B, K)


# speedup vs baseline: 2.1588x; 2.1588x over previous
import functools

import jax
import jax.numpy as jnp
from jax import lax
from jax.experimental import pallas as pl
from jax.experimental.pallas import tpu as pltpu
from jax.experimental.pallas import tpu_sc as plsc

NC = 2
NS = 16
NW = NC * NS
LANES = 16
BPR = 7


def _bitrev(x, nbits):
    r = 0
    for _ in range(nbits):
        r = (r << 1) | (x & 1)
        x >>= 1
    return r


_GDN = lax.GatherDimensionNumbers(
    offset_dims=(), collapsed_slice_dims=(0,), start_index_map=(0,))


def _dyn_gather(v, idx):
    return lax.gather(v, idx[:, None], dimension_numbers=_GDN,
                      slice_sizes=(1,),
                      mode=lax.GatherScatterMode.PROMISE_IN_BOUNDS)


def _perm_xor(v, lane, h):
    return _dyn_gather(v, lane ^ h)


def _butterfly(accs, lane):
    n = len(accs)
    assert n & (n - 1) == 0 and n <= LANES
    nbits = n.bit_length() - 1
    vecs = [accs[_bitrev(j, nbits)] for j in range(n)]
    h = LANES // 2
    while h >= n:
        vecs = [v + _perm_xor(v, lane, h) for v in vecs]
        h //= 2
    while len(vecs) > 1:
        nxt = []
        mask = (lane & h) == 0
        for i in range(0, len(vecs), 2):
            fx = vecs[i] + _perm_xor(vecs[i], lane, h)
            fy = vecs[i + 1] + _perm_xor(vecs[i + 1], lane, h)
            nxt.append(jnp.where(mask, fx, fy))
        vecs = nxt
        h //= 2
    return vecs[0]


def _make_kernel(B, K, D, BC):
    assert D == 100, "block/window constants assume D == 100"
    BPW = B // NW
    NCHUNK = BPW // BC
    R = BC * (K + 2)
    NG = R // 16

    n_left = 1 + (K - LANES)
    assert 0 < n_left <= 8

    mesh = plsc.VectorSubcoreMesh(core_axis_name="c", subcore_axis_name="s")

    @functools.partial(
        pl.kernel,
        compiler_params=pltpu.CompilerParams(use_tc_tiling_on_sc=False),
        out_type=[
            jax.ShapeDtypeStruct((B * LANES,), jnp.float32),
            jax.ShapeDtypeStruct((B * LANES,), jnp.float32),
        ],
        mesh=mesh,
        scratch_types=[
            pltpu.VMEM((R + 16,), jnp.int32),
            pltpu.VMEM((BPR * R,), jnp.int32),
            pltpu.VMEM((BPR * R, 16), jnp.float32),
            pltpu.VMEM((BC * LANES,), jnp.float32),
            pltpu.VMEM((BC * LANES,), jnp.float32),
            pltpu.SemaphoreType.DMA,
        ],
    )
    def k(c_hbm, p_hbm, n_hbm, wcb_hbm, wxb_hbm, neg16_out, left_out,
          iv, bidx, stage, neg_stage, left_stage, sem):
        wid = lax.axis_index("s") * NC + lax.axis_index("c")
        base_b = wid * BPW
        lane = lax.iota(jnp.int32, LANES)
        zero = jnp.zeros((LANES,), jnp.float32)

        def chunk_body(ci, carry):
            b0 = base_b + ci * BC
            pltpu.sync_copy(c_hbm.at[pl.ds(b0, BC)], iv.at[pl.ds(0, BC)])
            pltpu.sync_copy(p_hbm.at[pl.ds(b0, BC)], iv.at[pl.ds(BC, BC)])
            pltpu.sync_copy(n_hbm.at[pl.ds(b0 * K, BC * K)],
                            iv.at[pl.ds(2 * BC, BC * K)])

            for g in range(NG):
                idx16 = iv[pl.ds(g * 16, 16)]
                blk0 = (idx16 * 25) >> 2
                for t in range(BPR):
                    p = lane + (16 * t)
                    r = (p * 9363) >> 16
                    j = p - r * 7
                    bidx[pl.ds(g * 112 + 16 * t, 16)] = _dyn_gather(blk0, r) + j

            cps = [
                pltpu.async_copy(wcb_hbm.at[bidx.at[pl.ds(0, BPR * BC)]],
                                 stage.at[pl.ds(0, BPR * BC), :], sem),
                pltpu.async_copy(wxb_hbm.at[bidx.at[pl.ds(BPR * BC,
                                                          BPR * (R - BC))]],
                                 stage.at[pl.ds(BPR * BC, BPR * (R - BC)), :],
                                 sem),
            ]
            for cpy in cps:
                cpy.wait()

            def row_windows(r, tail_keep_mask):
                i = iv[pl.ds(r, 16)][0]
                o = (i * 4) & 15
                pidx = (lane + o) & 15
                fmask = lane < (16 - o)
                P = [_dyn_gather(stage[BPR * r + j], pidx) for j in range(BPR)]
                win = [jnp.where(fmask, P[w], P[w + 1]) for w in range(6)]
                w6 = P[6]
                if tail_keep_mask is not None:
                    w6 = jnp.where(tail_keep_mask, w6, 0.0)
                win.append(w6)
                return win

            tail_keep = lane < (D - 96)

            def b_body(b, carry2):
                cw = row_windows(b, tail_keep)

                def dot_acc(r):
                    xw = row_windows(r, None)
                    acc = cw[0] * xw[0]
                    for w in range(1, BPR):
                        acc = acc + cw[w] * xw[w]
                    return acc

                accs = [dot_acc(2 * BC + b * K + kk) for kk in range(LANES)]
                neg_stage[pl.ds(b * LANES, LANES)] = _butterfly(accs, lane)

                left = [dot_acc(BC + b)]
                for kk in range(LANES, K):
                    left.append(dot_acc(2 * BC + b * K + kk))
                while len(left) < 8:
                    left.append(zero)
                left_stage[pl.ds(b * LANES, LANES)] = _butterfly(left, lane)
                return carry2

            lax.fori_loop(0, BC, b_body, 0)
            pltpu.sync_copy(neg_stage,
                            neg16_out.at[pl.ds(b0 * LANES, BC * LANES)])
            pltpu.sync_copy(left_stage,
                            left_out.at[pl.ds(b0 * LANES, BC * LANES)])
            return carry

        lax.fori_loop(0, NCHUNK, chunk_body, 0)

    return k


def kernel(center, pos_context, neg_contexts, W_center, W_context):
    B = center.shape[0]
    K = neg_contexts.shape[1]
    D = W_center.shape[1]
    c = center.astype(jnp.int32)
    p = pos_context.astype(jnp.int32)
    n = neg_contexts.astype(jnp.int32).reshape(-1)
    wcb = W_center.astype(jnp.float32).reshape(-1, 16)
    wxb = W_context.astype(jnp.float32).reshape(-1, 16)

    k = _make_kernel(B, K, D, BC=32)
    neg16_flat, left_flat = k(c, p, n, wcb, wxb)
    neg16 = neg16_flat.reshape(B, LANES)
    left = left_flat.reshape(B, LANES)
    pos_dot = left[:, 0:1]
    neg_dot = jnp.concatenate([neg16, left[:, 1:1 + (K - LANES)]], axis=1)
    return pos_dot, neg_dot
